# baseline (speedup 1.0000x reference)
import functools
import jax
import jax.numpy as jnp
from jax import lax
from jax.experimental import pallas as pl
from jax.experimental.pallas import tpu as pltpu
from jax.experimental.pallas import tpu_sc as plsc

B, S, D, H, E, K = 2, 2048, 1024, 512, 8, 2
T = B * S
TK = T * K
EP = 128
M = 256
NT = TK // M + E
NROWS = NT * M

NC, NS = 2, 16
NW = NC * NS
RW = NROWS // NW
GC = 64
TW = T // NW
CC = 32


def _silu(x):
    return x * jax.nn.sigmoid(x)


def _shared_kernel(x_ref, wrp_ref, wgs_ref, wus_ref, wds_ref,
                   shared_ref, logits_ref):
    xt = x_ref[...]
    gate = lax.dot_general(xt, wgs_ref[...], (((1,), (1,)), ((), ())),
                           preferred_element_type=jnp.float32)
    up = lax.dot_general(xt, wus_ref[...], (((1,), (1,)), ((), ())),
                         preferred_element_type=jnp.float32)
    act = _silu(gate) * up
    shared_ref[...] = lax.dot_general(act, wds_ref[...],
                                      (((1,), (1,)), ((), ())),
                                      preferred_element_type=jnp.float32)
    logits_ref[...] = lax.dot_general(xt, wrp_ref[...],
                                      (((1,), (1,)), ((), ())),
                                      preferred_element_type=jnp.float32)


def _shared_and_logits(x, Wrp, Wg_s, Wu_s, Wd_s):
    return pl.pallas_call(
        _shared_kernel,
        grid=(T // M,),
        in_specs=[
            pl.BlockSpec((M, D), lambda t: (t, 0)),
            pl.BlockSpec((EP, D), lambda t: (0, 0)),
            pl.BlockSpec((H, D), lambda t: (0, 0)),
            pl.BlockSpec((H, D), lambda t: (0, 0)),
            pl.BlockSpec((D, H), lambda t: (0, 0)),
        ],
        out_specs=[
            pl.BlockSpec((M, D), lambda t: (t, 0)),
            pl.BlockSpec((M, EP), lambda t: (t, 0)),
        ],
        out_shape=[
            jax.ShapeDtypeStruct((T, D), jnp.float32),
            jax.ShapeDtypeStruct((T, EP), jnp.float32),
        ],
        compiler_params=pltpu.CompilerParams(
            dimension_semantics=("arbitrary",)),
    )(x, Wrp, Wg_s, Wu_s, Wd_s)


def _sc_gather_call(x, row_token):
    mesh = plsc.VectorSubcoreMesh(core_axis_name="c", subcore_axis_name="s")

    @functools.partial(
        pl.kernel, mesh=mesh,
        out_type=jax.ShapeDtypeStruct((NROWS, D), jnp.float32),
        scratch_types=[
            pltpu.VMEM((GC,), jnp.int32),
            pltpu.VMEM((GC, D), jnp.float32),
            pltpu.SemaphoreType.DMA,
        ],
    )
    def _gather(x_hbm, tok_hbm, xg_hbm, idx_v, rows_v, sem):
        wid = lax.axis_index("s") * NC + lax.axis_index("c")
        base = wid * RW

        def body(j, carry):
            off = base + j * GC
            pltpu.sync_copy(tok_hbm.at[pl.ds(off, GC)], idx_v)
            pltpu.async_copy(x_hbm.at[idx_v], rows_v, sem).wait()
            pltpu.sync_copy(rows_v, xg_hbm.at[pl.ds(off, GC)])
            return carry

        lax.fori_loop(0, RW // GC, body, 0)

    return _gather(x, row_token)


def _gmm_kernel(te_ref, xg_ref, w_ref, wg_ref, wu_ref, wd_ref, out_ref):
    xt = xg_ref[...]
    gate = lax.dot_general(xt, wg_ref[0], (((1,), (1,)), ((), ())),
                           preferred_element_type=jnp.float32)
    up = lax.dot_general(xt, wu_ref[0], (((1,), (1,)), ((), ())),
                         preferred_element_type=jnp.float32)
    act = _silu(gate) * up
    eo = lax.dot_general(act, wd_ref[0], (((1,), (1,)), ((), ())),
                         preferred_element_type=jnp.float32)
    out_ref[...] = eo * w_ref[...]


def _gmm_call(tile_expert, xg, row_weight, Wg_e, Wu_e, Wd_e):
    grid_spec = pltpu.PrefetchScalarGridSpec(
        num_scalar_prefetch=1,
        grid=(NT,),
        in_specs=[
            pl.BlockSpec((M, D), lambda i, te: (i, 0)),
            pl.BlockSpec((M, 1), lambda i, te: (i, 0)),
            pl.BlockSpec((1, H, D), lambda i, te: (te[i], 0, 0)),
            pl.BlockSpec((1, H, D), lambda i, te: (te[i], 0, 0)),
            pl.BlockSpec((1, D, H), lambda i, te: (te[i], 0, 0)),
        ],
        out_specs=pl.BlockSpec((M, D), lambda i, te: (i, 0)),
    )
    return pl.pallas_call(
        _gmm_kernel,
        grid_spec=grid_spec,
        out_shape=jax.ShapeDtypeStruct((NROWS, D), jnp.float32),
        compiler_params=pltpu.CompilerParams(
            dimension_semantics=("arbitrary",)),
    )(tile_expert, xg, row_weight, Wg_e, Wu_e, Wd_e)


def _sc_combine_call(shared, yg, posA, posB):
    mesh = plsc.VectorSubcoreMesh(core_axis_name="c", subcore_axis_name="s")

    @functools.partial(
        pl.kernel, mesh=mesh,
        out_type=jax.ShapeDtypeStruct((T, D), jnp.float32),
        scratch_types=[
            pltpu.VMEM((CC,), jnp.int32),
            pltpu.VMEM((CC,), jnp.int32),
            pltpu.VMEM((CC, D), jnp.float32),
            pltpu.VMEM((CC, D), jnp.float32),
            pltpu.VMEM((CC, D), jnp.float32),
            pltpu.SemaphoreType.DMA,
        ],
    )
    def _combine(shared_hbm, yg_hbm, posa_hbm, posb_hbm, out_hbm,
                 ia_v, ib_v, ya_v, yb_v, s_v, sem):
        wid = lax.axis_index("s") * NC + lax.axis_index("c")
        base = wid * TW

        def chunk(cidx, carry):
            off = base + cidx * CC
            pltpu.sync_copy(posa_hbm.at[pl.ds(off, CC)], ia_v)
            pltpu.sync_copy(posb_hbm.at[pl.ds(off, CC)], ib_v)
            pltpu.async_copy(yg_hbm.at[ia_v], ya_v, sem).wait()
            pltpu.async_copy(yg_hbm.at[ib_v], yb_v, sem).wait()
            pltpu.sync_copy(shared_hbm.at[pl.ds(off, CC)], s_v)

            def row(r, c2):
                def col(c, c3):
                    sl = pl.ds(c * 16, 16)
                    s_v[r, sl] = s_v[r, sl] + ya_v[r, sl] + yb_v[r, sl]
                    return c3
                return lax.fori_loop(0, D // 16, col, c2)

            lax.fori_loop(0, CC, row, 0)
            pltpu.sync_copy(s_v, out_hbm.at[pl.ds(off, CC)])
            return carry

        lax.fori_loop(0, TW // CC, chunk, 0)

    return _combine(shared, yg, posA, posB)


def kernel(hidden_states, Wr, Wg_s, Wu_s, Wd_s, Wg_e, Wu_e, Wd_e):
    b, s, d = hidden_states.shape
    x = hidden_states.reshape(T, d)
    Wrp = jnp.zeros((EP, d), jnp.float32).at[:E].set(Wr)

    shared, logits = _shared_and_logits(x, Wrp, Wg_s, Wu_s, Wd_s)

    probs = jax.nn.softmax(logits[:, :E].astype(jnp.float32), axis=-1)
    tkw, tki = lax.top_k(probs, K)
    tkw = tkw / jnp.sum(tkw, axis=-1, keepdims=True)
    flat_e = tki.reshape(-1).astype(jnp.int32)
    flat_w = tkw.reshape(-1).astype(jnp.float32)
    oh = (flat_e[:, None] == jnp.arange(E, dtype=jnp.int32)[None, :])
    ohi = oh.astype(jnp.int32)
    rank = jnp.sum((jnp.cumsum(ohi, axis=0) - 1) * ohi, axis=1)
    counts = jnp.sum(ohi, axis=0)
    tiles_e = (counts + M - 1) // M
    tile_end = jnp.cumsum(tiles_e)
    padded_start = (tile_end - tiles_e) * M
    dest = (padded_start[flat_e] + rank).astype(jnp.int32)
    arange_tk = jnp.arange(TK, dtype=jnp.int32)
    row_token = jnp.zeros((NROWS,), jnp.int32).at[dest].set(arange_tk // K)
    row_weight = jnp.zeros((NROWS, 1), jnp.float32).at[dest, 0].set(flat_w)
    tile_expert = jnp.searchsorted(
        tile_end, jnp.arange(NT, dtype=jnp.int32), side='right')
    tile_expert = jnp.minimum(tile_expert, E - 1).astype(jnp.int32)
    pos = dest.reshape(T, K)
    posA = pos[:, 0]
    posB = pos[:, 1]

    xg = _sc_gather_call(x, row_token)
    yg = _gmm_call(tile_expert, xg, row_weight, Wg_e, Wu_e, Wd_e)
    out = _sc_combine_call(shared, yg, posA, posB)
    return out.reshape(b, s, d)
